# Initial kernel scaffold; baseline (speedup 1.0000x reference)
import jax
import jax.numpy as jnp
from jax.experimental import pallas as pl


def kernel(src, x, blosum):
    raise NotImplementedError("write your pallas kernel here")


# speedup vs baseline: 1.6143x; 1.6143x over previous
import jax
import jax.numpy as jnp
from jax.experimental import pallas as pl
from jax.experimental.pallas import tpu as pltpu

_VOCAB = 28
_N_ALPHA = 20
_ALPHA_OFFSET = 3
_BLOSUM_DIM = 24


def _body(src_ref, x_ref, blosum_ref, out_ref):
    s = src_ref[0]
    valid = (s >= _ALPHA_OFFSET) & (s < _ALPHA_OFFSET + _N_ALPHA)
    idx = jnp.where(valid, s, _VOCAB - 1)
    ln = s.shape[0]
    iota = jax.lax.broadcasted_iota(jnp.int32, (ln, _VOCAB), 1)
    onehot = (iota == idx).astype(jnp.float32)
    coding = jnp.dot(onehot, blosum_ref[...],
                     preferred_element_type=jnp.float32)
    out_ref[0] = jnp.concatenate([x_ref[0], coding], axis=1)


def kernel(src, x, blosum):
    B, L, D = x.shape
    src3 = src.astype(jnp.int32).reshape(B, L, 1)
    out = pl.pallas_call(
        _body,
        grid=(B,),
        in_specs=[
            pl.BlockSpec((1, L, 1), lambda b: (b, 0, 0)),
            pl.BlockSpec((1, L, D), lambda b: (b, 0, 0)),
            pl.BlockSpec((_VOCAB, _BLOSUM_DIM), lambda b: (0, 0)),
        ],
        out_specs=pl.BlockSpec((1, L, D + _BLOSUM_DIM), lambda b: (b, 0, 0)),
        out_shape=jax.ShapeDtypeStruct((B, L, D + _BLOSUM_DIM), jnp.float32),
    )(src3, x, blosum)
    return out
